# baseline (speedup 1.0000x reference)
import functools

import jax
import jax.numpy as jnp
from jax import lax
from jax.experimental import pallas as pl
from jax.experimental.pallas import tpu as pltpu
from jax.experimental.pallas import tpu_sc as plsc

_CHUNK = 128
_NBUF = 4
_IDXBLK = 8


@functools.cache
def _build(n_rows: int, d: int, n_bins: int):
  info = plsc.get_sparse_core_info()
  nw = info.num_cores * info.num_subcores
  assert n_rows % (nw * _CHUNK * 2 * _IDXBLK) == 0
  assert n_bins % 8 == 0
  per_w = n_rows // nw
  n_chunks = per_w // _CHUNK
  n_blocks = n_chunks // _IDXBLK

  mesh = plsc.VectorSubcoreMesh(core_axis_name="c", subcore_axis_name="s")

  @functools.partial(
      pl.kernel,
      mesh=mesh,
      out_type=jax.ShapeDtypeStruct((n_rows, d), jnp.float32),
      scratch_types=[
          pltpu.VMEM((_IDXBLK, _CHUNK), jnp.int32),
          pltpu.VMEM((_IDXBLK, _CHUNK), jnp.int32),
          pltpu.VMEM((_NBUF * _CHUNK, d), jnp.float32),
          pltpu.VMEM_SHARED((info.num_subcores * n_bins, d), jnp.float32),
      ] + [pltpu.SemaphoreType.DMA] * (2 + 2 * _NBUF),
  )
  def emb(ids_hbm, table_hbm, out_hbm, idx0, idx1, rows, table_sh, *sems):
    sidx0, sidx1 = sems[0], sems[1]
    sg = sems[2:2 + _NBUF]
    ss = sems[2 + _NBUF:]

    sid = lax.axis_index("s")
    my_tbl = table_sh.at[pl.ds(sid * n_bins, n_bins)]
    pltpu.sync_copy(table_hbm, my_tbl)

    wid = lax.axis_index("s") * info.num_cores + lax.axis_index("c")
    out_base = wid * per_w
    id_base = wid * n_chunks
    last_id_row = id_base + n_chunks - _IDXBLK

    def rbuf(j):
      return rows.at[pl.ds(j * _CHUNK, _CHUNK)]

    def idx_start(dst, sem, id_row):
      r = jnp.minimum(id_row, last_id_row)
      pltpu.async_copy(ids_hbm.at[pl.ds(r, _IDXBLK)], dst, sem)

    def idx_wait(dst, sem):
      pltpu.make_async_copy(ids_hbm.at[pl.ds(id_base, _IDXBLK)], dst, sem).wait()

    def gsrc(j):
      return table_hbm if j == _IDXBLK - 1 else my_tbl

    def gather_start(idx, j, b):
      pltpu.async_copy(gsrc(j).at[idx.at[j]], rbuf(b), sg[b])

    def gather_wait(idx, j, b):
      pltpu.make_async_copy(gsrc(j).at[idx.at[j]], rbuf(b), sg[b]).wait()

    def scatter_start(j, off):
      pltpu.async_copy(rbuf(j), out_hbm.at[pl.ds(off, _CHUNK)], ss[j])

    def scatter_wait(j):
      pltpu.make_async_copy(rbuf(j), out_hbm.at[pl.ds(out_base, _CHUNK)],
                            ss[j]).wait()

    def block(u, idx_cur, idx_nxt, sem_nxt, first):
      base = out_base + u * (_IDXBLK * _CHUNK)
      idx_start(idx_nxt, sem_nxt, id_base + (u + 1) * _IDXBLK)
      for j in range(_IDXBLK):
        b = j % _NBUF
        if j >= _NBUF or not first:
          scatter_wait(b)
        gather_start(idx_cur, j, b)
        if j:
          gather_wait(idx_cur, j - 1, (j - 1) % _NBUF)
          scatter_start((j - 1) % _NBUF, base + (j - 1) * _CHUNK)
      gather_wait(idx_cur, _IDXBLK - 1, (_IDXBLK - 1) % _NBUF)
      scatter_start((_IDXBLK - 1) % _NBUF, base + (_IDXBLK - 1) * _CHUNK)
      idx_wait(idx_nxt, sem_nxt)

    def body(t, carry):
      block(2 * t, idx0, idx1, sidx1, first=False)
      block(2 * t + 1, idx1, idx0, sidx0, first=False)
      return carry

    pltpu.sync_copy(ids_hbm.at[pl.ds(id_base, _IDXBLK)], idx0)
    block(0, idx0, idx1, sidx1, first=True)
    block(1, idx1, idx0, sidx0, first=False)
    lax.fori_loop(1, n_blocks // 2, body, 0)
    for b in range(_NBUF):
      scatter_wait(b)

  return emb


def kernel(value_ids, value_floats, bin_emb_weight):
  del value_floats
  b, l = value_ids.shape
  n_bins, d = bin_emb_weight.shape
  n_rows = b * l
  n_pad = -(-n_bins // 8) * 8
  table_pad = jnp.pad(bin_emb_weight.astype(jnp.float32),
                      ((0, n_pad - n_bins), (0, 0)))
  ids2d = value_ids.reshape(n_rows // _CHUNK, _CHUNK).astype(jnp.int32)
  emb = _build(n_rows, d, n_pad)
  out = emb(ids2d, table_pad)
  return out.reshape(b, l, d)


# speedup vs baseline: 2.1988x; 2.1988x over previous
import functools

import jax
import jax.numpy as jnp
from jax import lax
from jax.experimental import pallas as pl
from jax.experimental.pallas import tpu as pltpu
from jax.experimental.pallas import tpu_sc as plsc

_CHUNK = 128
_NBUF = 4
_IDXBLK = 8


@functools.cache
def _build(n_rows: int, d: int, n_bins: int):
  info = plsc.get_sparse_core_info()
  nw = info.num_cores * info.num_subcores
  assert n_rows % (nw * _CHUNK * 2 * _IDXBLK) == 0
  assert n_bins % 8 == 0
  per_w = n_rows // nw
  n_chunks = per_w // _CHUNK
  n_blocks = n_chunks // _IDXBLK

  mesh = plsc.VectorSubcoreMesh(core_axis_name="c", subcore_axis_name="s")

  @functools.partial(
      pl.kernel,
      mesh=mesh,
      out_type=jax.ShapeDtypeStruct((n_rows, d), jnp.float32),
      scratch_types=[
          pltpu.VMEM((_IDXBLK, _CHUNK), jnp.int32),
          pltpu.VMEM((_IDXBLK, _CHUNK), jnp.int32),
          pltpu.VMEM((_NBUF * _CHUNK, d), jnp.float32),
          pltpu.VMEM_SHARED((info.num_subcores * n_bins, d), jnp.float32),
      ] + [pltpu.SemaphoreType.DMA] * (2 + 2 * _NBUF),
  )
  def emb(ids_hbm, table_hbm, out_hbm, idx0, idx1, rows, table_sh, *sems):
    sidx0, sidx1 = sems[0], sems[1]
    sg = sems[2:2 + _NBUF]
    ss = sems[2 + _NBUF:]

    sid = lax.axis_index("s")
    my_tbl = table_sh.at[pl.ds(sid * n_bins, n_bins)]
    pltpu.sync_copy(table_hbm, my_tbl)

    wid = lax.axis_index("s") * info.num_cores + lax.axis_index("c")
    out_base = wid * per_w
    id_base = wid * n_chunks
    last_id_row = id_base + n_chunks - _IDXBLK

    def rbuf(j):
      return rows.at[pl.ds(j * _CHUNK, _CHUNK)]

    def idx_start(dst, sem, id_row):
      r = jnp.minimum(id_row, last_id_row)
      pltpu.async_copy(ids_hbm.at[pl.ds(r, _IDXBLK)], dst, sem)

    def idx_wait(dst, sem):
      pltpu.make_async_copy(ids_hbm.at[pl.ds(id_base, _IDXBLK)], dst, sem).wait()

    def gather_start(idx, j, b):
      pltpu.async_copy(my_tbl.at[idx.at[j]], rbuf(b), sg[b])

    def gather_wait(idx, j, b):
      pltpu.make_async_copy(my_tbl.at[idx.at[j]], rbuf(b), sg[b]).wait()

    def scatter_start(j, off):
      pltpu.async_copy(rbuf(j), out_hbm.at[pl.ds(off, _CHUNK)], ss[j])

    def scatter_wait(j):
      pltpu.make_async_copy(rbuf(j), out_hbm.at[pl.ds(out_base, _CHUNK)],
                            ss[j]).wait()

    def block(u, idx_cur, idx_nxt, sem_nxt, first):
      base = out_base + u * (_IDXBLK * _CHUNK)
      idx_start(idx_nxt, sem_nxt, id_base + (u + 1) * _IDXBLK)
      for j in range(_IDXBLK):
        b = j % _NBUF
        if j >= _NBUF or not first:
          scatter_wait(b)
        gather_start(idx_cur, j, b)
        if j:
          gather_wait(idx_cur, j - 1, (j - 1) % _NBUF)
          scatter_start((j - 1) % _NBUF, base + (j - 1) * _CHUNK)
      gather_wait(idx_cur, _IDXBLK - 1, (_IDXBLK - 1) % _NBUF)
      scatter_start((_IDXBLK - 1) % _NBUF, base + (_IDXBLK - 1) * _CHUNK)
      idx_wait(idx_nxt, sem_nxt)

    def body(t, carry):
      block(2 * t, idx0, idx1, sidx1, first=False)
      block(2 * t + 1, idx1, idx0, sidx0, first=False)
      return carry

    pltpu.sync_copy(ids_hbm.at[pl.ds(id_base, _IDXBLK)], idx0)
    block(0, idx0, idx1, sidx1, first=True)
    block(1, idx1, idx0, sidx0, first=False)
    lax.fori_loop(1, n_blocks // 2, body, 0)
    for b in range(_NBUF):
      scatter_wait(b)

  return emb


def kernel(value_ids, value_floats, bin_emb_weight):
  del value_floats
  b, l = value_ids.shape
  n_bins, d = bin_emb_weight.shape
  n_rows = b * l
  n_pad = -(-n_bins // 8) * 8
  table_pad = jnp.pad(bin_emb_weight.astype(jnp.float32),
                      ((0, n_pad - n_bins), (0, 0)))
  ids2d = value_ids.reshape(n_rows // _CHUNK, _CHUNK).astype(jnp.int32)
  emb = _build(n_rows, d, n_pad)
  out = emb(ids2d, table_pad)
  return out.reshape(b, l, d)
